# baseline (speedup 1.0000x reference)
import functools

import jax
import jax.numpy as jnp
from jax import lax
from jax.experimental import pallas as pl
from jax.experimental.pallas import tpu as pltpu
from jax.experimental.pallas import tpu_sc as plsc

H = 1024
FF = 512
E = 8
ALPHA = 1.702
LIMIT = 7.0
T = 1024
BLK = 128
NT = 24
TS = NT * BLK
NW = 32
CHUNK = T // NW


def _router_body(x_ref, wr_ref, br_ref, pw_ref, pi_ref, te_ref):
    x = x_ref[...]
    logits = jnp.dot(x, wr_ref[...], preferred_element_type=jnp.float32)
    logits = logits + br_ref[...]
    m = jnp.max(logits, axis=1, keepdims=True)
    ex = jnp.exp(logits - m)
    probs = ex / jnp.sum(ex, axis=1, keepdims=True)
    eidx = lax.broadcasted_iota(jnp.int32, (T, E), 1)
    m1 = jnp.max(probs, axis=1, keepdims=True)
    a1 = jnp.min(jnp.where(probs >= m1, eidx, E), axis=1, keepdims=True)
    mask1 = eidx == a1
    probsb = jnp.where(mask1, -jnp.inf, probs)
    m2 = jnp.max(probsb, axis=1, keepdims=True)
    a2 = jnp.min(jnp.where(probsb >= m2, eidx, E), axis=1, keepdims=True)
    mask2 = eidx == a2
    s = m1 + m2 + 1e-20
    w0 = m1 / s
    w1 = m2 / s

    A = (mask1 | mask2).astype(jnp.bfloat16)
    ir = lax.broadcasted_iota(jnp.int32, (T, T), 0)
    ic = lax.broadcasted_iota(jnp.int32, (T, T), 1)
    tril = (ir > ic).astype(jnp.bfloat16)
    rank = jnp.dot(tril, A, preferred_element_type=jnp.float32)
    counts = jnp.sum(A.astype(jnp.float32), axis=0, keepdims=True)
    nblk = jnp.floor((counts + (BLK - 1)) * (1.0 / BLK))
    er = lax.broadcasted_iota(jnp.int32, (E, E), 0)
    ec = lax.broadcasted_iota(jnp.int32, (E, E), 1)
    triu = (er < ec).astype(jnp.float32)
    blkoff = jnp.dot(nblk, triu, preferred_element_type=jnp.float32)

    ti = lax.broadcasted_iota(jnp.int32, (NT, E), 0).astype(jnp.float32)
    te = jnp.sum((ti >= blkoff).astype(jnp.float32), axis=1, keepdims=True)
    te_ref[...] = te.astype(jnp.int32) - 1

    def slot(mask):
        maskf = mask.astype(jnp.float32)
        r = jnp.sum(maskf * rank, axis=1, keepdims=True)
        o = jnp.sum(maskf * blkoff, axis=1, keepdims=True)
        return o * BLK + r

    p0 = slot(mask1)
    p1 = slot(mask2)
    u = jnp.concatenate([p0, p1, w0, w1], axis=1)
    pw_ref[...] = u
    eye = (ir == ic).astype(jnp.float32)
    pwt = lax.dot_general(jnp.concatenate([p0, p1], axis=1), eye,
                          (((0,), (0,)), ((), ())),
                          precision=lax.Precision.HIGHEST,
                          preferred_element_type=jnp.float32)
    pi_ref[...] = pwt.astype(jnp.int32)


@jax.jit
def _router(flat, W_router, b_router):
    return pl.pallas_call(
        _router_body,
        grid=(1,),
        in_specs=[
            pl.BlockSpec((T, H), lambda i: (0, 0)),
            pl.BlockSpec((H, E), lambda i: (0, 0)),
            pl.BlockSpec((1, E), lambda i: (0, 0)),
        ],
        out_specs=[
            pl.BlockSpec((T, 4), lambda i: (0, 0)),
            pl.BlockSpec((2, T), lambda i: (0, 0)),
            pl.BlockSpec((NT, 1), lambda i: (0, 0)),
        ],
        out_shape=[
            jax.ShapeDtypeStruct((T, 4), jnp.float32),
            jax.ShapeDtypeStruct((2, T), jnp.int32),
            jax.ShapeDtypeStruct((NT, 1), jnp.int32),
        ],
    )(flat, W_router, b_router.reshape(1, E))


@functools.lru_cache(maxsize=None)
def _sc_kernels():
    mesh = plsc.VectorSubcoreMesh(core_axis_name="c", subcore_axis_name="s")
    nc = plsc.get_sparse_core_info().num_cores

    @functools.partial(
        pl.kernel,
        out_type=jax.ShapeDtypeStruct((TS, H), jnp.float32),
        mesh=mesh,
        scratch_types=[
            pltpu.VMEM((CHUNK,), jnp.int32),
            pltpu.VMEM((CHUNK,), jnp.int32),
            pltpu.VMEM((CHUNK, H), jnp.float32),
            pltpu.SemaphoreType.DMA,
        ],
    )
    def _dispatch(flat_hbm, p3d_hbm, xs_hbm, idx0_v, idx1_v, x_v, sem):
        w = lax.axis_index("s") * nc + lax.axis_index("c")
        pltpu.sync_copy(p3d_hbm.at[0, w], idx0_v)
        pltpu.sync_copy(p3d_hbm.at[1, w], idx1_v)
        pltpu.sync_copy(flat_hbm.at[pl.ds(w * CHUNK, CHUNK)], x_v)
        c0 = pltpu.async_copy(x_v, xs_hbm.at[idx0_v], sem)
        c1 = pltpu.async_copy(x_v, xs_hbm.at[idx1_v], sem)
        c0.wait()
        c1.wait()

    return _dispatch


def _group_body(te_ref, xs_ref, w1a_ref, w1b_ref, b1_ref, w2a_ref, w2b_ref,
                b2_ref, pw_ref, o_ref):
    i = pl.program_id(0)
    x = xs_ref[...]
    v = jnp.concatenate(
        [jnp.dot(x, w1a_ref[0], preferred_element_type=jnp.float32),
         jnp.dot(x, w1b_ref[0], preferred_element_type=jnp.float32)],
        axis=1) + b1_ref[0]
    gate = jnp.minimum(v, LIMIT)
    glu = gate * jax.nn.sigmoid(gate * ALPHA)
    up1 = jnp.clip(v, -LIMIT, LIMIT) + 1.0
    h_inter = glu * jnp.roll(up1, -1, axis=1)
    r = lax.broadcasted_iota(jnp.int32, (2 * FF, FF), 0)
    c = lax.broadcasted_iota(jnp.int32, (2 * FF, FF), 1)
    sel = (r == 2 * c).astype(jnp.float32)
    h = jnp.dot(h_inter, sel, preferred_element_type=jnp.float32)
    y = jnp.concatenate(
        [jnp.dot(h, w2a_ref[0], preferred_element_type=jnp.float32),
         jnp.dot(h, w2b_ref[0], preferred_element_type=jnp.float32)],
        axis=1) + b2_ref[0]
    y = jnp.where(jnp.isfinite(y), y, 0.0)
    slots = lax.broadcasted_iota(jnp.int32, (T, BLK), 1) + i * BLK
    s0 = pw_ref[:, 0:1].astype(jnp.int32)
    s1 = pw_ref[:, 1:2].astype(jnp.int32)
    S = (jnp.where(slots == s0, pw_ref[:, 2:3], 0.0)
         + jnp.where(slots == s1, pw_ref[:, 3:4], 0.0))
    contrib = jnp.dot(S, y, preferred_element_type=jnp.float32)

    @pl.when(i == 0)
    def _():
        o_ref[...] = contrib

    @pl.when(i > 0)
    def _():
        o_ref[...] = o_ref[...] + contrib


@jax.jit
def _grouped_mlp(tile_expert, xs, gate_up_proj, gate_up_proj_bias,
                 down_proj, down_proj_bias, pw):
    grid_spec = pltpu.PrefetchScalarGridSpec(
        num_scalar_prefetch=1,
        grid=(NT,),
        in_specs=[
            pl.BlockSpec((BLK, H), lambda i, te: (i, 0)),
            pl.BlockSpec((1, H, FF), lambda i, te: (te[i, 0], 0, 0)),
            pl.BlockSpec((1, H, FF), lambda i, te: (te[i, 0], 0, 1)),
            pl.BlockSpec((1, 1, 2 * FF), lambda i, te: (te[i, 0], 0, 0)),
            pl.BlockSpec((1, FF, H // 2), lambda i, te: (te[i, 0], 0, 0)),
            pl.BlockSpec((1, FF, H // 2), lambda i, te: (te[i, 0], 0, 1)),
            pl.BlockSpec((1, 1, H), lambda i, te: (te[i, 0], 0, 0)),
            pl.BlockSpec((T, 4), lambda i, te: (0, 0)),
        ],
        out_specs=pl.BlockSpec((T, H), lambda i, te: (0, 0)),
    )
    return pl.pallas_call(
        _group_body,
        grid_spec=grid_spec,
        out_shape=jax.ShapeDtypeStruct((T, H), jnp.float32),
    )(tile_expert, xs, gate_up_proj, gate_up_proj,
      gate_up_proj_bias.reshape(E, 1, 2 * FF), down_proj, down_proj,
      down_proj_bias.reshape(E, 1, H), pw)


def kernel(hidden_states, W_router, b_router, gate_up_proj,
           gate_up_proj_bias, down_proj, down_proj_bias):
    batch = hidden_states.shape[0]
    flat = hidden_states.reshape(T, H)
    pw, pi, tile_expert = _router(flat, W_router, b_router)
    p3d = pi.reshape(2, NW, CHUNK)
    dispatch = _sc_kernels()
    xs = dispatch(flat, p3d)
    out = _grouped_mlp(tile_expert, xs, gate_up_proj, gate_up_proj_bias,
                       down_proj, down_proj_bias, pw)
    return out.reshape(batch, -1, H)


# speedup vs baseline: 1.1660x; 1.1660x over previous
import functools

import jax
import jax.numpy as jnp
from jax import lax
from jax.experimental import pallas as pl
from jax.experimental.pallas import tpu as pltpu
from jax.experimental.pallas import tpu_sc as plsc

H = 1024
FF = 512
E = 8
ALPHA = 1.702
LIMIT = 7.0
T = 1024
BLK = 256
NT = 15
TS = NT * BLK
NW = 32
CHUNK = T // NW


def _router_body(x_ref, wr_ref, br_ref, pw_ref, pi_ref, te_ref):
    x = x_ref[...]
    logits = jnp.dot(x, wr_ref[...], preferred_element_type=jnp.float32)
    logits = logits + br_ref[...]
    m = jnp.max(logits, axis=1, keepdims=True)
    ex = jnp.exp(logits - m)
    probs = ex / jnp.sum(ex, axis=1, keepdims=True)
    eidx = lax.broadcasted_iota(jnp.int32, (T, E), 1)
    m1 = jnp.max(probs, axis=1, keepdims=True)
    a1 = jnp.min(jnp.where(probs >= m1, eidx, E), axis=1, keepdims=True)
    mask1 = eidx == a1
    probsb = jnp.where(mask1, -jnp.inf, probs)
    m2 = jnp.max(probsb, axis=1, keepdims=True)
    a2 = jnp.min(jnp.where(probsb >= m2, eidx, E), axis=1, keepdims=True)
    mask2 = eidx == a2
    s = m1 + m2 + 1e-20
    w0 = m1 / s
    w1 = m2 / s

    A = (mask1 | mask2).astype(jnp.bfloat16)
    ir = lax.broadcasted_iota(jnp.int32, (T, T), 0)
    ic = lax.broadcasted_iota(jnp.int32, (T, T), 1)
    tril = (ir > ic).astype(jnp.bfloat16)
    rank = jnp.dot(tril, A, preferred_element_type=jnp.float32)
    counts = jnp.sum(A.astype(jnp.float32), axis=0, keepdims=True)
    nblk = jnp.floor((counts + (BLK - 1)) * (1.0 / BLK))
    er = lax.broadcasted_iota(jnp.int32, (E, E), 0)
    ec = lax.broadcasted_iota(jnp.int32, (E, E), 1)
    triu = (er < ec).astype(jnp.float32)
    blkoff = jnp.dot(nblk, triu, preferred_element_type=jnp.float32)

    ti = lax.broadcasted_iota(jnp.int32, (NT, E), 0).astype(jnp.float32)
    te = jnp.sum((ti >= blkoff).astype(jnp.float32), axis=1, keepdims=True)
    te_ref[...] = te.astype(jnp.int32) - 1

    def slot(mask):
        maskf = mask.astype(jnp.float32)
        r = jnp.sum(maskf * rank, axis=1, keepdims=True)
        o = jnp.sum(maskf * blkoff, axis=1, keepdims=True)
        return o * BLK + r

    p0 = slot(mask1)
    p1 = slot(mask2)
    u = jnp.concatenate([p0, p1, w0, w1], axis=1)
    eye = (ir == ic).astype(jnp.float32)
    pwt = lax.dot_general(u, eye, (((0,), (0,)), ((), ())),
                          precision=lax.Precision.HIGHEST,
                          preferred_element_type=jnp.float32)
    pw_ref[...] = pwt
    pi_ref[...] = pwt[0:2, :].astype(jnp.int32)


@jax.jit
def _router(flat, W_router, b_router):
    return pl.pallas_call(
        _router_body,
        grid=(1,),
        in_specs=[
            pl.BlockSpec((T, H), lambda i: (0, 0)),
            pl.BlockSpec((H, E), lambda i: (0, 0)),
            pl.BlockSpec((1, E), lambda i: (0, 0)),
        ],
        out_specs=[
            pl.BlockSpec((4, T), lambda i: (0, 0)),
            pl.BlockSpec((2, T), lambda i: (0, 0)),
            pl.BlockSpec((NT, 1), lambda i: (0, 0)),
        ],
        out_shape=[
            jax.ShapeDtypeStruct((4, T), jnp.float32),
            jax.ShapeDtypeStruct((2, T), jnp.int32),
            jax.ShapeDtypeStruct((NT, 1), jnp.int32),
        ],
    )(flat, W_router, b_router.reshape(1, E))


@functools.lru_cache(maxsize=None)
def _sc_kernels():
    mesh = plsc.VectorSubcoreMesh(core_axis_name="c", subcore_axis_name="s")
    nc = plsc.get_sparse_core_info().num_cores

    @functools.partial(
        pl.kernel,
        out_type=jax.ShapeDtypeStruct((TS, H), jnp.float32),
        mesh=mesh,
        scratch_types=[
            pltpu.VMEM((CHUNK,), jnp.int32),
            pltpu.VMEM((CHUNK,), jnp.int32),
            pltpu.VMEM((CHUNK, H), jnp.float32),
            pltpu.SemaphoreType.DMA,
        ],
    )
    def _dispatch(flat_hbm, p3d_hbm, xs_hbm, idx0_v, idx1_v, x_v, sem):
        w = lax.axis_index("s") * nc + lax.axis_index("c")
        pltpu.sync_copy(p3d_hbm.at[0, w], idx0_v)
        pltpu.sync_copy(p3d_hbm.at[1, w], idx1_v)
        pltpu.sync_copy(flat_hbm.at[pl.ds(w * CHUNK, CHUNK)], x_v)
        c0 = pltpu.async_copy(x_v, xs_hbm.at[idx0_v], sem)
        c1 = pltpu.async_copy(x_v, xs_hbm.at[idx1_v], sem)
        c0.wait()
        c1.wait()

    @functools.partial(
        pl.kernel,
        out_type=jax.ShapeDtypeStruct((T, H), jnp.float32),
        mesh=mesh,
        scratch_types=[
            pltpu.VMEM((CHUNK,), jnp.int32),
            pltpu.VMEM((CHUNK,), jnp.int32),
            pltpu.VMEM((CHUNK, H), jnp.float32),
            pltpu.VMEM((CHUNK, H), jnp.float32),
            pltpu.SemaphoreType.DMA,
        ],
    )
    def _combine(ys_hbm, p3d_hbm, out_hbm, idx0_v, idx1_v, y0_v, y1_v, sem):
        w = lax.axis_index("s") * nc + lax.axis_index("c")
        pltpu.sync_copy(p3d_hbm.at[0, w], idx0_v)
        pltpu.sync_copy(p3d_hbm.at[1, w], idx1_v)
        c0 = pltpu.async_copy(ys_hbm.at[idx0_v], y0_v, sem)
        c1 = pltpu.async_copy(ys_hbm.at[idx1_v], y1_v, sem)
        c0.wait()
        c1.wait()

        def row(r, carry):
            for cc in range(0, H, 16 * 8):
                for u in range(8):
                    sl = pl.ds(cc + u * 16, 16)
                    y0_v[r, sl] = y0_v[r, sl] + y1_v[r, sl]
            return carry

        lax.fori_loop(0, CHUNK, row, 0)
        pltpu.sync_copy(y0_v, out_hbm.at[pl.ds(w * CHUNK, CHUNK)])

    return _dispatch, _combine


def _group_body(te_ref, xs_ref, w1a_ref, w1b_ref, b1_ref, w2a_ref, w2b_ref,
                b2_ref, pw_ref, y_ref):
    i = pl.program_id(0)
    x = xs_ref[...]
    v = jnp.concatenate(
        [jnp.dot(x, w1a_ref[0], preferred_element_type=jnp.float32),
         jnp.dot(x, w1b_ref[0], preferred_element_type=jnp.float32)],
        axis=1) + b1_ref[0]
    gate = jnp.minimum(v, LIMIT)
    glu = gate * jax.nn.sigmoid(gate * ALPHA)
    up1 = jnp.clip(v, -LIMIT, LIMIT) + 1.0
    h_inter = glu * jnp.roll(up1, -1, axis=1)
    r = lax.broadcasted_iota(jnp.int32, (2 * FF, FF), 0)
    c = lax.broadcasted_iota(jnp.int32, (2 * FF, FF), 1)
    sel = (r == 2 * c).astype(jnp.float32)
    h = jnp.dot(h_inter, sel, preferred_element_type=jnp.float32)
    y = jnp.concatenate(
        [jnp.dot(h, w2a_ref[0], preferred_element_type=jnp.float32),
         jnp.dot(h, w2b_ref[0], preferred_element_type=jnp.float32)],
        axis=1) + b2_ref[0]
    rows = lax.broadcasted_iota(jnp.int32, (BLK, T), 0) + i * BLK
    hit0 = rows == pw_ref[0:1, :].astype(jnp.int32)
    hit1 = rows == pw_ref[1:2, :].astype(jnp.int32)
    wrow = (jnp.sum(jnp.where(hit0, pw_ref[2:3, :], 0.0), axis=1,
                    keepdims=True)
            + jnp.sum(jnp.where(hit1, pw_ref[3:4, :], 0.0), axis=1,
                      keepdims=True))
    y_ref[...] = wrow * y


@jax.jit
def _grouped_mlp(tile_expert, xs, gate_up_proj, gate_up_proj_bias,
                 down_proj, down_proj_bias, pw):
    grid_spec = pltpu.PrefetchScalarGridSpec(
        num_scalar_prefetch=1,
        grid=(NT,),
        in_specs=[
            pl.BlockSpec((BLK, H), lambda i, te: (i, 0)),
            pl.BlockSpec((1, H, FF), lambda i, te: (te[i, 0], 0, 0)),
            pl.BlockSpec((1, H, FF), lambda i, te: (te[i, 0], 0, 1)),
            pl.BlockSpec((1, 1, 2 * FF), lambda i, te: (te[i, 0], 0, 0)),
            pl.BlockSpec((1, FF, H // 2), lambda i, te: (te[i, 0], 0, 0)),
            pl.BlockSpec((1, FF, H // 2), lambda i, te: (te[i, 0], 0, 1)),
            pl.BlockSpec((1, 1, H), lambda i, te: (te[i, 0], 0, 0)),
            pl.BlockSpec((4, T), lambda i, te: (0, 0)),
        ],
        out_specs=pl.BlockSpec((BLK, H), lambda i, te: (i, 0)),
    )
    return pl.pallas_call(
        _group_body,
        grid_spec=grid_spec,
        out_shape=jax.ShapeDtypeStruct((TS, H), jnp.float32),
    )(tile_expert, xs, gate_up_proj, gate_up_proj,
      gate_up_proj_bias.reshape(E, 1, 2 * FF), down_proj, down_proj,
      down_proj_bias.reshape(E, 1, H), pw)


def kernel(hidden_states, W_router, b_router, gate_up_proj,
           gate_up_proj_bias, down_proj, down_proj_bias):
    batch = hidden_states.shape[0]
    flat = hidden_states.reshape(T, H)
    pw, pi, tile_expert = _router(flat, W_router, b_router)
    p3d = pi.reshape(2, NW, CHUNK)
    dispatch, combine = _sc_kernels()
    xs = dispatch(flat, p3d)
    ys = _grouped_mlp(tile_expert, xs, gate_up_proj, gate_up_proj_bias,
                      down_proj, down_proj_bias, pw)
    out = combine(ys, p3d)
    return out.reshape(batch, -1, H)
